# baseline (speedup 1.0000x reference)
import functools

import jax
import jax.numpy as jnp
from jax.experimental import pallas as pl

_S_BLK = 256


def _tsoftmax_body(x_ref, w_ref, o_ref):
    w = w_ref[...]
    x = x_ref[...]
    m = jnp.max(w, axis=1, keepdims=True)
    e = jnp.exp(w - m)
    z = jnp.sum(e, axis=1)
    num = jnp.sum(e * x, axis=1)
    o_ref[...] = num / z


@jax.jit
def kernel(inputs, weights):
    b, s, i, j = weights.shape
    n = b * s
    x = inputs.reshape(n, i, 1)
    w = weights.reshape(n, i, j)
    grid = (n // _S_BLK,)
    out = pl.pallas_call(
        _tsoftmax_body,
        grid=grid,
        in_specs=[
            pl.BlockSpec((_S_BLK, i, 1), lambda g: (g, 0, 0)),
            pl.BlockSpec((_S_BLK, i, j), lambda g: (g, 0, 0)),
        ],
        out_specs=pl.BlockSpec((_S_BLK, j), lambda g: (g, 0)),
        out_shape=jax.ShapeDtypeStruct((n, j), jnp.float32),
    )(x, w)
    return out.reshape(b, s, j)
